# baseline (speedup 1.0000x reference)
import jax
import jax.numpy as jnp
from jax import lax
from jax.experimental import pallas as pl
from jax.experimental.pallas import tpu as pltpu
from jax.experimental.pallas import tpu_sc as plsc

N = 10000
E = 320000
DF = 128
DO = 10
B = 100
HW = 16
HW3 = 3 * HW
NC, NS, L = 2, 16, 16
NW = NC * NS
NPAD = 10240
RPS = NPAD // NS


def _make_sc_edge(with_g, with_ex, use_orows, CH):
    ntab = 9 if with_g else 6
    NCHUNK = E // CH
    NPAIR = (-(-NCHUNK // NW) + 1) // 2

    def body(*refs):
        (sd_hbm, earr_hbm, hext_hbm, sdg_hbm, zero_hbm), refs = refs[:5], refs[5:]
        if with_ex:
            (ex_hbm, acc0_hbm, acc1_hbm), refs = refs[:3], refs[3:]
        else:
            (acc0_hbm, acc1_hbm), refs = refs[:2], refs[2:]
        (tabs_v, sd_v0, sd_v1, sdst_v0, sdst_v1, earr_v0, earr_v1), refs = refs[:7], refs[7:]
        if with_ex:
            (exs_v0, exs_v1), refs = refs[:2], refs[2:]
            exss = (exs_v0, exs_v1)
        (rows_v0, rows_v1), refs = refs[:2], refs[2:]
        rowss = (rows_v0, rows_v1)
        if use_orows:
            (orows_v0, orows_v1), refs = refs[:2], refs[2:]
            orowss = (orows_v0, orows_v1)
        else:
            orowss = rowss
        (acc_sp, sin0, sin1, sg0, sg1, ss0, ss1), refs = refs[:7], refs[7:]
        if with_ex:
            (se0, se1), refs = refs[:2], refs[2:]
            ses = (se0, se1)
        assert not refs
        sds = (sd_v0, sd_v1)
        sdsts = (sdst_v0, sdst_v1)
        earrs = (earr_v0, earr_v1)
        sins = (sin0, sin1)
        sgs = (sg0, sg1)
        sss = (ss0, ss1)

        cid = lax.axis_index("c")
        sid = lax.axis_index("s")
        wid = sid * NC + cid

        pltpu.sync_copy(sdg_hbm, tabs_v)
        rsl = pl.ds(sid * RPS, RPS)
        pltpu.sync_copy(zero_hbm.at[rsl], acc_sp.at[rsl])
        plsc.subcore_barrier()

        def ci_of(j):
            return wid + j * NW

        def valid(j):
            return jnp.logical_and(j >= 0, ci_of(j) < NCHUNK)

        def esl_of(j):
            return pl.ds(ci_of(j) * CH, CH)

        def in_copies(j, b):
            esl = esl_of(j)
            return [(sd_hbm.at[:, esl], sds[b]), (earr_hbm.at[:, esl], earrs[b])]

        def fire_in(j, b):
            @pl.when(valid(j))
            def _():
                for s_, d_ in in_copies(j, b):
                    pltpu.async_copy(s_, d_, sins[b])

        def wait_in(j, b):
            @pl.when(valid(j))
            def _():
                for s_, d_ in in_copies(j, b):
                    pltpu.make_async_copy(s_, d_, sins[b]).wait()

        def fire_gather(j, b):
            @pl.when(valid(j))
            def _():
                pltpu.async_copy(hext_hbm.at[sds[b].at[0]], rowss[b], sgs[b])

        def wait_gather(j, b):
            @pl.when(valid(j))
            def _():
                pltpu.make_async_copy(hext_hbm.at[sds[b].at[0]], rowss[b], sgs[b]).wait()

        def compute(j, b):
            @pl.when(valid(j))
            def _():
                for g in range(CH // L):
                    gsl = pl.ds(g * L, L)
                    si = sds[b][0, gsl]
                    di = sds[b][1, gsl]
                    sdsts[b][gsl] = di
                    for c in range(3):
                        s16 = plsc.load_gather(tabs_v.at[c], [si])
                        d16 = plsc.load_gather(tabs_v.at[3 + c], [di])
                        et = earrs[b][c, gsl]
                        if with_g:
                            g16 = plsc.load_gather(tabs_v.at[6 + c], [di])
                            lg = s16 + d16 + et * g16
                        else:
                            lg = s16 + d16 + et
                        lg = jnp.where(lg >= 0.0, lg, 0.2 * lg)
                        ex16 = jnp.exp(lg)
                        if with_ex:
                            exss[b][c, gsl] = ex16
                        csl = pl.ds(c * HW, HW)
                        for jj in range(L):
                            i = g * L + jj
                            orowss[b][i, csl] = rowss[b][i, csl] * ex16[jj]

        def fire_out(j, b):
            @pl.when(valid(j))
            def _():
                pltpu.async_copy(orowss[b], acc_sp.at[sdsts[b]], sss[b], add=True)
                if with_ex:
                    pltpu.async_copy(exss[b], ex_hbm.at[:, esl_of(j)], ses[b])

        def wait_out(j, b):
            @pl.when(valid(j))
            def _():
                pltpu.make_async_copy(orowss[b], acc_sp.at[sdsts[b]], sss[b]).wait()
                if with_ex:
                    pltpu.make_async_copy(exss[b], ex_hbm.at[:, esl_of(j)], ses[b]).wait()

        def step(j, b):
            nb = 1 - b
            wait_gather(j, b)
            wait_out(j - 1, nb)
            wait_in(j + 1, nb)
            fire_gather(j + 1, nb)
            compute(j, b)
            fire_out(j, b)
            fire_in(j + 2, b)

        fire_in(0, 0)
        fire_in(1, 1)
        wait_in(0, 0)
        fire_gather(0, 0)

        def pair(t, carry):
            j = t * 2
            step(j, 0)
            step(j + 1, 1)
            return carry

        lax.fori_loop(0, NPAIR, pair, 0)
        plsc.subcore_barrier()

        @pl.when(cid == 0)
        def _():
            pltpu.sync_copy(acc_sp.at[rsl], acc0_hbm.at[rsl])

        @pl.when(cid == 1)
        def _():
            pltpu.sync_copy(acc_sp.at[rsl], acc1_hbm.at[rsl])

    out_type = []
    if with_ex:
        out_type.append(jax.ShapeDtypeStruct((3, E), jnp.float32))
    out_type += [jax.ShapeDtypeStruct((NPAD, HW3), jnp.float32)] * 2

    scratch = [pltpu.VMEM((ntab, N), jnp.float32)]
    scratch += [pltpu.VMEM((2, CH), jnp.int32)] * 2
    scratch += [pltpu.VMEM((CH,), jnp.int32)] * 2
    scratch += [pltpu.VMEM((3, CH), jnp.float32)] * 2
    if with_ex:
        scratch += [pltpu.VMEM((3, CH), jnp.float32)] * 2
    scratch += [pltpu.VMEM((CH, HW3), jnp.float32)] * 2
    if use_orows:
        scratch += [pltpu.VMEM((CH, HW3), jnp.float32)] * 2
    scratch += [pltpu.VMEM_SHARED((NPAD, HW3), jnp.float32)]
    scratch += [pltpu.SemaphoreType.DMA] * (8 if with_ex else 6)

    return pl.kernel(
        body,
        out_type=tuple(out_type),
        mesh=plsc.VectorSubcoreMesh(core_axis_name="c", subcore_axis_name="s"),
        compiler_params=pltpu.CompilerParams(
            needs_layout_passes=False, use_tc_tiling_on_sc=False),
        scratch_types=scratch,
    )


_sc_edge_l1 = _make_sc_edge(with_g=False, with_ex=True, use_orows=True, CH=32)
_sc_edge_l2 = _make_sc_edge(with_g=True, with_ex=False, use_orows=True, CH=32)


def _hext_of(h3):
    parts = []
    for c in range(3):
        parts.append(h3[:, c * DO:(c + 1) * DO])
        parts.append(jnp.ones((N, 1), jnp.float32))
        parts.append(jnp.zeros((N, HW - DO - 1), jnp.float32))
    return jnp.concatenate(parts, axis=1)


def _prep1_body(x_ref, w3_ref, a1_ref, aev_ref, ea3_ref,
                hext_ref, sdg_ref, earr_ref):
    h3 = jnp.dot(x_ref[...], w3_ref[...], preferred_element_type=jnp.float32)
    hext_ref[...] = _hext_of(h3)
    sdg_ref[...] = jnp.dot(h3, a1_ref[...], preferred_element_type=jnp.float32)
    earr_ref[...] = ea3_ref[...] * aev_ref[...]


_prep1 = pl.pallas_call(
    _prep1_body,
    out_shape=(
        jax.ShapeDtypeStruct((N, HW3), jnp.float32),
        jax.ShapeDtypeStruct((N, 6), jnp.float32),
        jax.ShapeDtypeStruct((3, E), jnp.float32),
    ),
)


def _prep2_body(a0_ref, a1_ref, w3_ref, a2_ref, aev_ref, hext_ref, sdg_ref):
    acc = a0_ref[:N, :] + a1_ref[:N, :]
    x1s, gs = [], []
    for c in range(3):
        num = acc[:, c * HW:c * HW + DO]
        den = acc[:, c * HW + DO:c * HW + DO + 1]
        invden = 1.0 / (den + 1e-16)
        x1s.append(num * invden)
        gs.append(invden * aev_ref[0, c])
    x1 = jnp.concatenate(x1s, axis=1)
    h3 = jnp.dot(x1, w3_ref[...], preferred_element_type=jnp.float32)
    hext_ref[...] = _hext_of(h3)
    sd = jnp.dot(h3, a2_ref[...], preferred_element_type=jnp.float32)
    sdg_ref[...] = jnp.concatenate([sd] + gs, axis=1)


_prep2 = pl.pallas_call(
    _prep2_body,
    out_shape=(
        jax.ShapeDtypeStruct((N, HW3), jnp.float32),
        jax.ShapeDtypeStruct((N, 9), jnp.float32),
    ),
)


def _combine_body(a0_ref, a1_ref, out_ref):
    acc = a0_ref[:N, :] + a1_ref[:N, :]
    for c in range(3):
        num = acc[:, c * HW:c * HW + DO]
        den = acc[:, c * HW + DO:c * HW + DO + 1]
        out_ref[pl.ds(c * N, N), :] = num / (den + 1e-16)


_combine = pl.pallas_call(
    _combine_body,
    out_shape=jax.ShapeDtypeStruct((3 * N, DO), jnp.float32),
)


def _head_body(h_ref, w1_ref, b1_ref, w2_ref, b2_ref, o_ref):
    a = jnp.maximum(
        jnp.dot(h_ref[...], w1_ref[...], preferred_element_type=jnp.float32)
        + b1_ref[...], 0.0)
    o_ref[...] = jnp.dot(a, w2_ref[...], preferred_element_type=jnp.float32) + b2_ref[...]


_head = pl.pallas_call(
    _head_body,
    out_shape=jax.ShapeDtypeStruct((B, 2), jnp.float32),
)


def _block_diag_attn(ps, key_src, key_dst):
    a = jnp.zeros((3 * DO, 6), jnp.float32)
    for c in range(3):
        a = a.at[c * DO:(c + 1) * DO, c].set(ps[c][key_src])
        a = a.at[c * DO:(c + 1) * DO, 3 + c].set(ps[c][key_dst])
    return a


def kernel(x, edge_index, edge_attr, y, params):
    sd = edge_index.astype(jnp.int32)
    ea3 = jnp.transpose(edge_attr[:, :3])
    zero_acc = jnp.zeros((NPAD, HW3), jnp.float32)

    p1 = [params['c%d_1' % c] for c in range(3)]
    p2 = [params['c%d_2' % c] for c in range(3)]
    w3_1 = jnp.concatenate([p['W'] for p in p1], axis=1)
    a1 = _block_diag_attn(p1, 'a_src', 'a_dst')
    aev1 = jnp.stack([p['a_e'][0] for p in p1])[:, None]
    w3_2 = jax.scipy.linalg.block_diag(*[p['W'] for p in p2])
    a2 = _block_diag_attn(p2, 'a_src', 'a_dst')
    aev2 = jnp.stack([p['a_e'][0] for p in p2])[None, :]

    hext1, sdg1, earr1 = _prep1(x, w3_1, a1, aev1, ea3)
    ex1, a10, a11 = _sc_edge_l1(sd, earr1, hext1,
                                jnp.transpose(sdg1), zero_acc)
    hext2, sdg2 = _prep2(a10, a11, w3_2, a2, aev2)
    a20, a21 = _sc_edge_l2(sd, ex1, hext2,
                           jnp.transpose(sdg2), zero_acc)
    h3 = _combine(a20, a21)
    h = h3.reshape(B, -1)
    out = _head(h, params['fc1_w'], params['fc1_b'][None, :],
                params['fc2_w'], params['fc2_b'][None, :])
    return out


# speedup vs baseline: 1.3642x; 1.3642x over previous
import jax
import jax.numpy as jnp
from jax import lax
from jax.experimental import pallas as pl
from jax.experimental.pallas import tpu as pltpu
from jax.experimental.pallas import tpu_sc as plsc

N = 10000
E = 320000
DF = 128
DO = 10
B = 100
HW = 16
HW3 = 3 * HW
NC, NS, L = 2, 16, 16
NW = NC * NS
NPAD = 10240
RPS = NPAD // NS


def _make_sc_edge(with_g, with_ex, use_orows, CH):
    ntab = 9 if with_g else 6
    NCHUNK = E // CH
    NPAIR = (-(-NCHUNK // NW) + 1) // 2

    def body(*refs):
        (sd_hbm, earr_hbm, hext_hbm, sdg_hbm, zero_hbm), refs = refs[:5], refs[5:]
        if with_ex:
            (ex_hbm, acc0_hbm, acc1_hbm), refs = refs[:3], refs[3:]
        else:
            (acc0_hbm, acc1_hbm), refs = refs[:2], refs[2:]
        (tabs_v, sd_v0, sd_v1, sdst_v0, sdst_v1, earr_v0, earr_v1), refs = refs[:7], refs[7:]
        if with_ex:
            (exs_v0, exs_v1), refs = refs[:2], refs[2:]
            exss = (exs_v0, exs_v1)
        (rows_v0, rows_v1), refs = refs[:2], refs[2:]
        rowss = (rows_v0, rows_v1)
        if use_orows:
            (orows_v0, orows_v1), refs = refs[:2], refs[2:]
            orowss = (orows_v0, orows_v1)
        else:
            orowss = rowss
        (acc_sp, sin0, sin1, sg0, sg1, ss0, ss1), refs = refs[:7], refs[7:]
        if with_ex:
            (se0, se1), refs = refs[:2], refs[2:]
            ses = (se0, se1)
        assert not refs
        sds = (sd_v0, sd_v1)
        sdsts = (sdst_v0, sdst_v1)
        earrs = (earr_v0, earr_v1)
        sins = (sin0, sin1)
        sgs = (sg0, sg1)
        sss = (ss0, ss1)

        cid = lax.axis_index("c")
        sid = lax.axis_index("s")
        wid = sid * NC + cid

        pltpu.sync_copy(sdg_hbm, tabs_v)
        rsl = pl.ds(sid * RPS, RPS)
        pltpu.sync_copy(zero_hbm.at[rsl], acc_sp.at[rsl])
        plsc.subcore_barrier()

        def ci_of(j):
            return wid + j * NW

        def valid(j):
            return jnp.logical_and(j >= 0, ci_of(j) < NCHUNK)

        def esl_of(j):
            return pl.ds(ci_of(j) * CH, CH)

        def in_copies(j, b):
            esl = esl_of(j)
            return [(sd_hbm.at[:, esl], sds[b]), (earr_hbm.at[:, esl], earrs[b])]

        def fire_in(j, b):
            @pl.when(valid(j))
            def _():
                for s_, d_ in in_copies(j, b):
                    pltpu.async_copy(s_, d_, sins[b])

        def wait_in(j, b):
            @pl.when(valid(j))
            def _():
                for s_, d_ in in_copies(j, b):
                    pltpu.make_async_copy(s_, d_, sins[b]).wait()

        def fire_gather(j, b):
            @pl.when(valid(j))
            def _():
                pltpu.async_copy(hext_hbm.at[sds[b].at[0]], rowss[b], sgs[b])

        def wait_gather(j, b):
            @pl.when(valid(j))
            def _():
                pltpu.make_async_copy(hext_hbm.at[sds[b].at[0]], rowss[b], sgs[b]).wait()

        def compute(j, b):
            @pl.when(valid(j))
            def _():
                for g in range(CH // L):
                    gsl = pl.ds(g * L, L)
                    si = sds[b][0, gsl]
                    di = sds[b][1, gsl]
                    sdsts[b][gsl] = di
                    for c in range(3):
                        s16 = plsc.load_gather(tabs_v.at[c], [si])
                        d16 = plsc.load_gather(tabs_v.at[3 + c], [di])
                        et = earrs[b][c, gsl]
                        if with_g:
                            g16 = plsc.load_gather(tabs_v.at[6 + c], [di])
                            lg = s16 + d16 + et * g16
                        else:
                            lg = s16 + d16 + et
                        lg = jnp.where(lg >= 0.0, lg, 0.2 * lg)
                        ex16 = jnp.exp(lg)
                        if with_ex:
                            exss[b][c, gsl] = ex16
                        csl = pl.ds(c * HW, HW)
                        for jj in range(L):
                            i = g * L + jj
                            orowss[b][i, csl] = rowss[b][i, csl] * ex16[jj]

        def fire_out(j, b):
            @pl.when(valid(j))
            def _():
                pltpu.async_copy(orowss[b], acc_sp.at[sdsts[b]], sss[b], add=True)
                if with_ex:
                    pltpu.async_copy(exss[b], ex_hbm.at[:, esl_of(j)], ses[b])

        def wait_out(j, b):
            @pl.when(valid(j))
            def _():
                pltpu.make_async_copy(orowss[b], acc_sp.at[sdsts[b]], sss[b]).wait()
                if with_ex:
                    pltpu.make_async_copy(exss[b], ex_hbm.at[:, esl_of(j)], ses[b]).wait()

        def step(j, b):
            nb = 1 - b
            wait_gather(j, b)
            wait_out(j - 1, nb)
            wait_in(j + 1, nb)
            fire_gather(j + 1, nb)
            compute(j, b)
            fire_out(j, b)
            fire_in(j + 2, b)

        fire_in(0, 0)
        fire_in(1, 1)
        wait_in(0, 0)
        fire_gather(0, 0)

        def pair(t, carry):
            j = t * 2
            step(j, 0)
            step(j + 1, 1)
            return carry

        lax.fori_loop(0, NPAIR, pair, 0)
        plsc.subcore_barrier()

        @pl.when(cid == 0)
        def _():
            pltpu.sync_copy(acc_sp.at[rsl], acc0_hbm.at[rsl])

        @pl.when(cid == 1)
        def _():
            pltpu.sync_copy(acc_sp.at[rsl], acc1_hbm.at[rsl])

    out_type = []
    if with_ex:
        out_type.append(jax.ShapeDtypeStruct((3, E), jnp.float32))
    out_type += [jax.ShapeDtypeStruct((NPAD, HW3), jnp.float32)] * 2

    scratch = [pltpu.VMEM((ntab, N), jnp.float32)]
    scratch += [pltpu.VMEM((2, CH), jnp.int32)] * 2
    scratch += [pltpu.VMEM((CH,), jnp.int32)] * 2
    scratch += [pltpu.VMEM((3, CH), jnp.float32)] * 2
    if with_ex:
        scratch += [pltpu.VMEM((3, CH), jnp.float32)] * 2
    scratch += [pltpu.VMEM((CH, HW3), jnp.float32)] * 2
    if use_orows:
        scratch += [pltpu.VMEM((CH, HW3), jnp.float32)] * 2
    scratch += [pltpu.VMEM_SHARED((NPAD, HW3), jnp.float32)]
    scratch += [pltpu.SemaphoreType.DMA] * (8 if with_ex else 6)

    return pl.kernel(
        body,
        out_type=tuple(out_type),
        mesh=plsc.VectorSubcoreMesh(core_axis_name="c", subcore_axis_name="s"),
        compiler_params=pltpu.CompilerParams(
            needs_layout_passes=False, use_tc_tiling_on_sc=False),
        scratch_types=scratch,
    )


_sc_edge_l1 = _make_sc_edge(with_g=False, with_ex=True, use_orows=True, CH=64)
_sc_edge_l2 = _make_sc_edge(with_g=True, with_ex=False, use_orows=False, CH=64)


def _hext_of(h3):
    parts = []
    for c in range(3):
        parts.append(h3[:, c * DO:(c + 1) * DO])
        parts.append(jnp.ones((N, 1), jnp.float32))
        parts.append(jnp.zeros((N, HW - DO - 1), jnp.float32))
    return jnp.concatenate(parts, axis=1)


def _prep1_body(x_ref, w3_ref, a1_ref, aev_ref, ea3_ref,
                hext_ref, sdg_ref, earr_ref):
    h3 = jnp.dot(x_ref[...], w3_ref[...], preferred_element_type=jnp.float32)
    hext_ref[...] = _hext_of(h3)
    sdg_ref[...] = jnp.dot(h3, a1_ref[...], preferred_element_type=jnp.float32)
    earr_ref[...] = ea3_ref[...] * aev_ref[...]


_prep1 = pl.pallas_call(
    _prep1_body,
    out_shape=(
        jax.ShapeDtypeStruct((N, HW3), jnp.float32),
        jax.ShapeDtypeStruct((N, 6), jnp.float32),
        jax.ShapeDtypeStruct((3, E), jnp.float32),
    ),
)


def _prep2_body(a0_ref, a1_ref, w3_ref, a2_ref, aev_ref, hext_ref, sdg_ref):
    acc = a0_ref[:N, :] + a1_ref[:N, :]
    x1s, gs = [], []
    for c in range(3):
        num = acc[:, c * HW:c * HW + DO]
        den = acc[:, c * HW + DO:c * HW + DO + 1]
        invden = 1.0 / (den + 1e-16)
        x1s.append(num * invden)
        gs.append(invden * aev_ref[0, c])
    x1 = jnp.concatenate(x1s, axis=1)
    h3 = jnp.dot(x1, w3_ref[...], preferred_element_type=jnp.float32)
    hext_ref[...] = _hext_of(h3)
    sd = jnp.dot(h3, a2_ref[...], preferred_element_type=jnp.float32)
    sdg_ref[...] = jnp.concatenate([sd] + gs, axis=1)


_prep2 = pl.pallas_call(
    _prep2_body,
    out_shape=(
        jax.ShapeDtypeStruct((N, HW3), jnp.float32),
        jax.ShapeDtypeStruct((N, 9), jnp.float32),
    ),
)


def _combine_body(a0_ref, a1_ref, out_ref):
    acc = a0_ref[:N, :] + a1_ref[:N, :]
    for c in range(3):
        num = acc[:, c * HW:c * HW + DO]
        den = acc[:, c * HW + DO:c * HW + DO + 1]
        out_ref[pl.ds(c * N, N), :] = num / (den + 1e-16)


_combine = pl.pallas_call(
    _combine_body,
    out_shape=jax.ShapeDtypeStruct((3 * N, DO), jnp.float32),
)


def _head_body(h_ref, w1_ref, b1_ref, w2_ref, b2_ref, o_ref):
    a = jnp.maximum(
        jnp.dot(h_ref[...], w1_ref[...], preferred_element_type=jnp.float32)
        + b1_ref[...], 0.0)
    o_ref[...] = jnp.dot(a, w2_ref[...], preferred_element_type=jnp.float32) + b2_ref[...]


_head = pl.pallas_call(
    _head_body,
    out_shape=jax.ShapeDtypeStruct((B, 2), jnp.float32),
)


def _block_diag_attn(ps, key_src, key_dst):
    a = jnp.zeros((3 * DO, 6), jnp.float32)
    for c in range(3):
        a = a.at[c * DO:(c + 1) * DO, c].set(ps[c][key_src])
        a = a.at[c * DO:(c + 1) * DO, 3 + c].set(ps[c][key_dst])
    return a


def kernel(x, edge_index, edge_attr, y, params):
    sd = edge_index.astype(jnp.int32)
    ea3 = jnp.transpose(edge_attr[:, :3])
    zero_acc = jnp.zeros((NPAD, HW3), jnp.float32)

    p1 = [params['c%d_1' % c] for c in range(3)]
    p2 = [params['c%d_2' % c] for c in range(3)]
    w3_1 = jnp.concatenate([p['W'] for p in p1], axis=1)
    a1 = _block_diag_attn(p1, 'a_src', 'a_dst')
    aev1 = jnp.stack([p['a_e'][0] for p in p1])[:, None]
    w3_2 = jax.scipy.linalg.block_diag(*[p['W'] for p in p2])
    a2 = _block_diag_attn(p2, 'a_src', 'a_dst')
    aev2 = jnp.stack([p['a_e'][0] for p in p2])[None, :]

    hext1, sdg1, earr1 = _prep1(x, w3_1, a1, aev1, ea3)
    ex1, a10, a11 = _sc_edge_l1(sd, earr1, hext1,
                                jnp.transpose(sdg1), zero_acc)
    hext2, sdg2 = _prep2(a10, a11, w3_2, a2, aev2)
    a20, a21 = _sc_edge_l2(sd, ex1, hext2,
                           jnp.transpose(sdg2), zero_acc)
    h3 = _combine(a20, a21)
    h = h3.reshape(B, -1)
    out = _head(h, params['fc1_w'], params['fc1_b'][None, :],
                params['fc2_w'], params['fc2_b'][None, :])
    return out


# speedup vs baseline: 1.3698x; 1.0041x over previous
import jax
import jax.numpy as jnp
from jax import lax
from jax.experimental import pallas as pl
from jax.experimental.pallas import tpu as pltpu
from jax.experimental.pallas import tpu_sc as plsc

N = 10000
E = 320000
DF = 128
DO = 10
B = 100
HW = 16
HW3 = 3 * HW
NC, NS, L = 2, 16, 16
NW = NC * NS
NPAD = 10240
RPS = NPAD // NS


def _make_sc_edge(with_g, with_ex, use_orows, CH):
    ntab = 9 if with_g else 6
    NCHUNK = E // CH
    NPAIR = (-(-NCHUNK // NW) + 1) // 2

    def body(*refs):
        (sd_hbm, earr_hbm, hext_hbm, sdg_hbm, zero_hbm), refs = refs[:5], refs[5:]
        if with_ex:
            (ex_hbm, acc0_hbm, acc1_hbm), refs = refs[:3], refs[3:]
        else:
            (acc0_hbm, acc1_hbm), refs = refs[:2], refs[2:]
        (tabs_v, sd_v0, sd_v1, sdst_v0, sdst_v1, earr_v0, earr_v1), refs = refs[:7], refs[7:]
        if with_ex:
            (exs_v0, exs_v1), refs = refs[:2], refs[2:]
            exss = (exs_v0, exs_v1)
        (rows_v0, rows_v1), refs = refs[:2], refs[2:]
        rowss = (rows_v0, rows_v1)
        if use_orows:
            (orows_v0, orows_v1), refs = refs[:2], refs[2:]
            orowss = (orows_v0, orows_v1)
        else:
            orowss = rowss
        (acc_sp, sin0, sin1, sg0, sg1, ss0, ss1), refs = refs[:7], refs[7:]
        if with_ex:
            (se0, se1), refs = refs[:2], refs[2:]
            ses = (se0, se1)
        assert not refs
        sds = (sd_v0, sd_v1)
        sdsts = (sdst_v0, sdst_v1)
        earrs = (earr_v0, earr_v1)
        sins = (sin0, sin1)
        sgs = (sg0, sg1)
        sss = (ss0, ss1)

        cid = lax.axis_index("c")
        sid = lax.axis_index("s")
        wid = sid * NC + cid

        rsl = pl.ds(sid * RPS, RPS)
        pltpu.async_copy(sdg_hbm, tabs_v, sin0)
        pltpu.async_copy(zero_hbm.at[rsl], acc_sp.at[rsl], sg0)
        pltpu.make_async_copy(sdg_hbm, tabs_v, sin0).wait()
        pltpu.make_async_copy(zero_hbm.at[rsl], acc_sp.at[rsl], sg0).wait()
        plsc.subcore_barrier()

        def ci_of(j):
            return wid + j * NW

        def valid(j):
            return jnp.logical_and(j >= 0, ci_of(j) < NCHUNK)

        def esl_of(j):
            return pl.ds(ci_of(j) * CH, CH)

        def in_copies(j, b):
            esl = esl_of(j)
            return [(sd_hbm.at[:, esl], sds[b]), (earr_hbm.at[:, esl], earrs[b])]

        def fire_in(j, b):
            @pl.when(valid(j))
            def _():
                for s_, d_ in in_copies(j, b):
                    pltpu.async_copy(s_, d_, sins[b])

        def wait_in(j, b):
            @pl.when(valid(j))
            def _():
                for s_, d_ in in_copies(j, b):
                    pltpu.make_async_copy(s_, d_, sins[b]).wait()

        def fire_gather(j, b):
            @pl.when(valid(j))
            def _():
                pltpu.async_copy(hext_hbm.at[sds[b].at[0]], rowss[b], sgs[b])

        def wait_gather(j, b):
            @pl.when(valid(j))
            def _():
                pltpu.make_async_copy(hext_hbm.at[sds[b].at[0]], rowss[b], sgs[b]).wait()

        def compute(j, b):
            @pl.when(valid(j))
            def _():
                for g in range(CH // L):
                    gsl = pl.ds(g * L, L)
                    si = sds[b][0, gsl]
                    di = sds[b][1, gsl]
                    sdsts[b][gsl] = di
                    for c in range(3):
                        s16 = plsc.load_gather(tabs_v.at[c], [si])
                        d16 = plsc.load_gather(tabs_v.at[3 + c], [di])
                        et = earrs[b][c, gsl]
                        if with_g:
                            g16 = plsc.load_gather(tabs_v.at[6 + c], [di])
                            lg = s16 + d16 + et * g16
                        else:
                            lg = s16 + d16 + et
                        lg = jnp.where(lg >= 0.0, lg, 0.2 * lg)
                        ex16 = jnp.exp(lg)
                        if with_ex:
                            exss[b][c, gsl] = ex16
                        csl = pl.ds(c * HW, HW)
                        for jj in range(L):
                            i = g * L + jj
                            orowss[b][i, csl] = rowss[b][i, csl] * ex16[jj]

        def fire_out(j, b):
            @pl.when(valid(j))
            def _():
                pltpu.async_copy(orowss[b], acc_sp.at[sdsts[b]], sss[b], add=True)
                if with_ex:
                    pltpu.async_copy(exss[b], ex_hbm.at[:, esl_of(j)], ses[b])

        def wait_out(j, b):
            @pl.when(valid(j))
            def _():
                pltpu.make_async_copy(orowss[b], acc_sp.at[sdsts[b]], sss[b]).wait()
                if with_ex:
                    pltpu.make_async_copy(exss[b], ex_hbm.at[:, esl_of(j)], ses[b]).wait()

        def step(j, b):
            nb = 1 - b
            wait_gather(j, b)
            wait_out(j - 1, nb)
            wait_in(j + 1, nb)
            fire_gather(j + 1, nb)
            compute(j, b)
            fire_out(j, b)
            fire_in(j + 2, b)

        fire_in(0, 0)
        fire_in(1, 1)
        wait_in(0, 0)
        fire_gather(0, 0)

        def pair(t, carry):
            j = t * 2
            step(j, 0)
            step(j + 1, 1)
            return carry

        lax.fori_loop(0, NPAIR, pair, 0)
        plsc.subcore_barrier()

        @pl.when(cid == 0)
        def _():
            pltpu.sync_copy(acc_sp.at[rsl], acc0_hbm.at[rsl])

        @pl.when(cid == 1)
        def _():
            pltpu.sync_copy(acc_sp.at[rsl], acc1_hbm.at[rsl])

    out_type = []
    if with_ex:
        out_type.append(jax.ShapeDtypeStruct((3, E), jnp.float32))
    out_type += [jax.ShapeDtypeStruct((NPAD, HW3), jnp.float32)] * 2

    scratch = [pltpu.VMEM((ntab, N), jnp.float32)]
    scratch += [pltpu.VMEM((2, CH), jnp.int32)] * 2
    scratch += [pltpu.VMEM((CH,), jnp.int32)] * 2
    scratch += [pltpu.VMEM((3, CH), jnp.float32)] * 2
    if with_ex:
        scratch += [pltpu.VMEM((3, CH), jnp.float32)] * 2
    scratch += [pltpu.VMEM((CH, HW3), jnp.float32)] * 2
    if use_orows:
        scratch += [pltpu.VMEM((CH, HW3), jnp.float32)] * 2
    scratch += [pltpu.VMEM_SHARED((NPAD, HW3), jnp.float32)]
    scratch += [pltpu.SemaphoreType.DMA] * (8 if with_ex else 6)

    return pl.kernel(
        body,
        out_type=tuple(out_type),
        mesh=plsc.VectorSubcoreMesh(core_axis_name="c", subcore_axis_name="s"),
        compiler_params=pltpu.CompilerParams(
            needs_layout_passes=False, use_tc_tiling_on_sc=False),
        scratch_types=scratch,
    )


_sc_edge_l1 = _make_sc_edge(with_g=False, with_ex=True, use_orows=True, CH=64)
_sc_edge_l2 = _make_sc_edge(with_g=True, with_ex=False, use_orows=False, CH=64)


def _hext_of(h3):
    parts = []
    for c in range(3):
        parts.append(h3[:, c * DO:(c + 1) * DO])
        parts.append(jnp.ones((N, 1), jnp.float32))
        parts.append(jnp.zeros((N, HW - DO - 1), jnp.float32))
    return jnp.concatenate(parts, axis=1)


def _prep1_body(x_ref, w3_ref, a1_ref, aev_ref, ea3_ref,
                hext_ref, sdg_ref, earr_ref):
    h3 = jnp.dot(x_ref[...], w3_ref[...], preferred_element_type=jnp.float32)
    hext_ref[...] = _hext_of(h3)
    sdg_ref[...] = jnp.dot(h3, a1_ref[...], preferred_element_type=jnp.float32)
    earr_ref[...] = ea3_ref[...] * aev_ref[...]


_prep1 = pl.pallas_call(
    _prep1_body,
    out_shape=(
        jax.ShapeDtypeStruct((N, HW3), jnp.float32),
        jax.ShapeDtypeStruct((N, 6), jnp.float32),
        jax.ShapeDtypeStruct((3, E), jnp.float32),
    ),
)


def _prep2_body(a0_ref, a1_ref, w3_ref, a2_ref, aev_ref, hext_ref, sdg_ref):
    acc = a0_ref[:N, :] + a1_ref[:N, :]
    x1s, gs = [], []
    for c in range(3):
        num = acc[:, c * HW:c * HW + DO]
        den = acc[:, c * HW + DO:c * HW + DO + 1]
        invden = 1.0 / (den + 1e-16)
        x1s.append(num * invden)
        gs.append(invden * aev_ref[0, c])
    x1 = jnp.concatenate(x1s, axis=1)
    h3 = jnp.dot(x1, w3_ref[...], preferred_element_type=jnp.float32)
    hext_ref[...] = _hext_of(h3)
    sd = jnp.dot(h3, a2_ref[...], preferred_element_type=jnp.float32)
    sdg_ref[...] = jnp.concatenate([sd] + gs, axis=1)


_prep2 = pl.pallas_call(
    _prep2_body,
    out_shape=(
        jax.ShapeDtypeStruct((N, HW3), jnp.float32),
        jax.ShapeDtypeStruct((N, 9), jnp.float32),
    ),
)


def _combine_body(a0_ref, a1_ref, out_ref):
    acc = a0_ref[:N, :] + a1_ref[:N, :]
    for c in range(3):
        num = acc[:, c * HW:c * HW + DO]
        den = acc[:, c * HW + DO:c * HW + DO + 1]
        out_ref[pl.ds(c * N, N), :] = num / (den + 1e-16)


_combine = pl.pallas_call(
    _combine_body,
    out_shape=jax.ShapeDtypeStruct((3 * N, DO), jnp.float32),
)


def _head_body(h_ref, w1_ref, b1_ref, w2_ref, b2_ref, o_ref):
    a = jnp.maximum(
        jnp.dot(h_ref[...], w1_ref[...], preferred_element_type=jnp.float32)
        + b1_ref[...], 0.0)
    o_ref[...] = jnp.dot(a, w2_ref[...], preferred_element_type=jnp.float32) + b2_ref[...]


_head = pl.pallas_call(
    _head_body,
    out_shape=jax.ShapeDtypeStruct((B, 2), jnp.float32),
)


def _block_diag_attn(ps, key_src, key_dst):
    a = jnp.zeros((3 * DO, 6), jnp.float32)
    for c in range(3):
        a = a.at[c * DO:(c + 1) * DO, c].set(ps[c][key_src])
        a = a.at[c * DO:(c + 1) * DO, 3 + c].set(ps[c][key_dst])
    return a


def kernel(x, edge_index, edge_attr, y, params):
    sd = edge_index.astype(jnp.int32)
    ea3 = jnp.transpose(edge_attr[:, :3])
    zero_acc = jnp.zeros((NPAD, HW3), jnp.float32)

    p1 = [params['c%d_1' % c] for c in range(3)]
    p2 = [params['c%d_2' % c] for c in range(3)]
    w3_1 = jnp.concatenate([p['W'] for p in p1], axis=1)
    a1 = _block_diag_attn(p1, 'a_src', 'a_dst')
    aev1 = jnp.stack([p['a_e'][0] for p in p1])[:, None]
    w3_2 = jax.scipy.linalg.block_diag(*[p['W'] for p in p2])
    a2 = _block_diag_attn(p2, 'a_src', 'a_dst')
    aev2 = jnp.stack([p['a_e'][0] for p in p2])[None, :]

    hext1, sdg1, earr1 = _prep1(x, w3_1, a1, aev1, ea3)
    ex1, a10, a11 = _sc_edge_l1(sd, earr1, hext1,
                                jnp.transpose(sdg1), zero_acc)
    hext2, sdg2 = _prep2(a10, a11, w3_2, a2, aev2)
    a20, a21 = _sc_edge_l2(sd, ex1, hext2,
                           jnp.transpose(sdg2), zero_acc)
    h3 = _combine(a20, a21)
    h = h3.reshape(B, -1)
    out = _head(h, params['fc1_w'], params['fc1_b'][None, :],
                params['fc2_w'], params['fc2_b'][None, :])
    return out
